# baseline (speedup 1.0000x reference)
import functools

import jax
import jax.numpy as jnp
from jax import lax
from jax.experimental import pallas as pl
from jax.experimental.pallas import tpu as pltpu
from jax.experimental.pallas import tpu_sc as plsc

_NC = 2
_NS = 16
_L = 16
_NW = _NC * _NS

_YEARS = 14
_YEAR0 = 2010
_EMB = 128

_MD = 12 * 31
_WH = 7 * 24
_YM = _YEARS * 6
_ROWS = _MD + _WH + _YM

_CHUNK = 128
_NF = 6


def _combined_rows(year, month, day, wday, hour, minute):
    in_range = (year >= _YEAR0) & (year <= _YEAR0 + _YEARS - 1)
    yi = jnp.where(in_range, year - _YEAR0, year)
    yi = jnp.minimum(jnp.maximum(yi, 0), _YEARS - 1)
    mi = jnp.minimum(jnp.maximum(month - 1, 0), 11)
    di = jnp.minimum(jnp.maximum(day - 1, 0), 30)
    wi = jnp.minimum(jnp.maximum(wday, 0), 6)
    hi = jnp.minimum(jnp.maximum(hour, 0), 23)
    ni = jnp.minimum(jnp.maximum(lax.div(minute, 10), 0), 5)
    md = (mi * 31 + di) * _EMB
    wh = (wi * 24 + hi + _MD) * _EMB
    ym = (yi * 6 + ni + _MD + _WH) * _EMB
    return md, wh, ym


def _build_pairs(tbl_v, a_v, b_v, dst_off, nb, count):
    @plsc.parallel_loop(0, count, 1, unroll=2)
    def body(r):
        i = r // nb
        j = r - i * nb
        for k in range(_EMB // _L):
            va = a_v[pl.ds(i * _EMB + k * _L, _L)]
            vb = b_v[pl.ds(j * _EMB + k * _L, _L)]
            tbl_v[pl.ds((dst_off + r) * _EMB + k * _L, _L)] = va + vb


def _sc_lookup(xt_flat, npos, yw, mw, dw, wdw, hw, nw):
    per_w = npos // _NW
    nchunk = per_w // _CHUNK
    mesh = plsc.VectorSubcoreMesh(core_axis_name="c", subcore_axis_name="s")

    @functools.partial(
        pl.kernel,
        mesh=mesh,
        compiler_params=pltpu.CompilerParams(needs_layout_passes=False),
        out_type=jax.ShapeDtypeStruct((npos * _EMB,), jnp.float32),
        scratch_types=[
            pltpu.VMEM((_ROWS * _EMB,), jnp.float32),
            pltpu.VMEM((_YEARS * _EMB,), jnp.float32),
            pltpu.VMEM((12 * _EMB,), jnp.float32),
            pltpu.VMEM((31 * _EMB,), jnp.float32),
            pltpu.VMEM((7 * _EMB,), jnp.float32),
            pltpu.VMEM((24 * _EMB,), jnp.float32),
            pltpu.VMEM((6 * _EMB,), jnp.float32),
            pltpu.VMEM((_NF * _CHUNK,), jnp.int32),
            pltpu.VMEM((_NF * _CHUNK,), jnp.int32),
            pltpu.VMEM((_CHUNK * _EMB,), jnp.float32),
            pltpu.VMEM((_CHUNK * _EMB,), jnp.float32),
            pltpu.SemaphoreType.DMA,
            pltpu.SemaphoreType.DMA,
            pltpu.SemaphoreType.DMA,
            pltpu.SemaphoreType.DMA,
        ],
    )
    def body(xt_hbm, yw_hbm, mw_hbm, dw_hbm, wdw_hbm, hw_hbm, nw_hbm,
             out_hbm, tbl_v, yv, mv, dv, wv, hv, nv, xa_v, xb_v, oa_v, ob_v,
             sem_xa, sem_xb, sem_oa, sem_ob):
        wid = lax.axis_index("s") * _NC + lax.axis_index("c")
        base = wid * per_w

        def x_copies(t, x_v, sem):
            pos0 = base + t * _CHUNK
            return [pltpu.make_async_copy(
                        xt_hbm.at[pl.ds(f * npos + pos0, _CHUNK)],
                        x_v.at[pl.ds(f * _CHUNK, _CHUNK)], sem)
                    for f in range(_NF)]

        def o_copy(t, o_v, sem):
            return pltpu.make_async_copy(
                o_v, out_hbm.at[pl.ds((base + t * _CHUNK) * _EMB,
                                      _CHUNK * _EMB)], sem)

        for c in x_copies(0, xa_v, sem_xa):
            c.start()

        tbl_loads = [pltpu.make_async_copy(src, dst, sem_oa)
                     for src, dst in ((yw_hbm, yv), (mw_hbm, mv),
                                      (dw_hbm, dv), (wdw_hbm, wv),
                                      (hw_hbm, hv), (nw_hbm, nv))]
        for c in tbl_loads:
            c.start()
        for c in tbl_loads:
            c.wait()

        _build_pairs(tbl_v, mv, dv, 0, 31, _MD)
        _build_pairs(tbl_v, wv, hv, _MD, 24, _WH)
        _build_pairs(tbl_v, yv, nv, _MD + _WH, 6, _YM)

        lane = lax.iota(jnp.int32, _L)
        zero = jnp.int32(0)

        def fill_uniform(o_v, y0, m0, d0, w0, h0, n0):
            md, wh, ym = _combined_rows(y0, m0, d0, w0, h0, n0)
            rows = [tbl_v[pl.ds(md + k * _L, _L)]
                    + tbl_v[pl.ds(wh + k * _L, _L)]
                    + tbl_v[pl.ds(ym + k * _L, _L)]
                    for k in range(_EMB // _L)]

            @plsc.parallel_loop(0, _CHUNK, 1, unroll=4)
            def fill_body(p):
                for k in range(_EMB // _L):
                    o_v[pl.ds(p * _EMB + k * _L, _L)] = rows[k]

        def fill_general(x_v, o_v):
            for g in range(_CHUNK // _L):
                md, wh, ym = _combined_rows(
                    x_v[pl.ds(0 * _CHUNK + g * _L, _L)],
                    x_v[pl.ds(1 * _CHUNK + g * _L, _L)],
                    x_v[pl.ds(2 * _CHUNK + g * _L, _L)],
                    x_v[pl.ds(3 * _CHUNK + g * _L, _L)],
                    x_v[pl.ds(4 * _CHUNK + g * _L, _L)],
                    x_v[pl.ds(5 * _CHUNK + g * _L, _L)])
                ob = (g * _L + lane) * _EMB

                @plsc.parallel_loop(0, _EMB, 1, unroll=8)
                def col_body(c):
                    val = (plsc.load_gather(tbl_v, [md + c])
                           + plsc.load_gather(tbl_v, [wh + c])
                           + plsc.load_gather(tbl_v, [ym + c]))
                    plsc.store_scatter(o_v, [ob + c], val)

        def process(t, x_v, x_nxt, sem_nxt, o_v, sem_o, carry):
            y0p, m0p, d0p, w0p, h0p, n0p, valid = carry

            @pl.when(t + 1 < nchunk)
            def _():
                for c in x_copies(t + 1, x_nxt, sem_nxt):
                    c.start()

            heads = [x_v[pl.ds(f * _CHUNK, _L)] for f in range(_NF)]
            y0 = heads[0][0]
            m0 = heads[1][0]
            d0 = heads[2][0]
            w0 = heads[3][0]
            h0 = heads[4][0]
            n0 = heads[5][0]
            scalars = (y0, m0, d0, w0, h0, n0)
            acc = lane < _L
            for f in range(_NF):
                acc = acc & (heads[f] == scalars[f])
            for g in range(1, _CHUNK // _L):
                for f in range(_NF):
                    acc = acc & (x_v[pl.ds(f * _CHUNK + g * _L, _L)]
                                 == scalars[f])
            uniform = jnp.all(acc)
            same = (uniform & (valid == 1)
                    & (y0 == y0p) & (m0 == m0p) & (d0 == d0p)
                    & (w0 == w0p) & (h0 == h0p) & (n0 == n0p))

            @pl.when(t > 1)
            def _():
                o_copy(t - 2, o_v, sem_o).wait()

            def stale():
                lax.cond(uniform,
                         lambda: fill_uniform(o_v, y0, m0, d0, w0, h0, n0),
                         lambda: fill_general(x_v, o_v))

            lax.cond(same, lambda: None, stale)
            o_copy(t, o_v, sem_o).start()
            return (y0, m0, d0, w0, h0, n0,
                    jnp.where(uniform, jnp.int32(1), zero))

        def pair_body(i, carry):
            ca, cb = carry
            t = i * 2
            for c in x_copies(t, xa_v, sem_xa):
                c.wait()
            ca = process(t, xa_v, xb_v, sem_xb, oa_v, sem_oa, ca)
            for c in x_copies(t + 1, xb_v, sem_xb):
                c.wait()
            cb = process(t + 1, xb_v, xa_v, sem_xa, ob_v, sem_ob, cb)
            return (ca, cb)

        slot0 = (zero, zero, zero, zero, zero, zero, zero)
        lax.fori_loop(0, nchunk // 2, pair_body, (slot0, slot0))
        o_copy(nchunk - 2, oa_v, sem_oa).wait()
        o_copy(nchunk - 1, ob_v, sem_ob).wait()

    return body(xt_flat, yw, mw, dw, wdw, hw, nw)


def kernel(x, year_w, month_w, day_w, weekday_w, hour_w, min_w):
    b, l, _ = x.shape
    xt_flat = jnp.transpose(x, (2, 0, 1)).reshape(-1)
    out_flat = _sc_lookup(
        xt_flat, b * l,
        year_w.reshape(-1), month_w.reshape(-1), day_w.reshape(-1),
        weekday_w.reshape(-1), hour_w.reshape(-1), min_w.reshape(-1),
    )
    return out_flat.reshape(b, l, _EMB)


# speedup vs baseline: 1.1009x; 1.1009x over previous
import functools

import jax
import jax.numpy as jnp
from jax import lax
from jax.experimental import pallas as pl
from jax.experimental.pallas import tpu as pltpu
from jax.experimental.pallas import tpu_sc as plsc

_NC = 2
_NS = 16
_L = 16
_NW = _NC * _NS

_YEARS = 14
_YEAR0 = 2010
_EMB = 128

_MD = 12 * 31
_WH = 7 * 24
_YM = _YEARS * 6
_ROWS = _MD + _WH + _YM

_CHUNK = 256
_NF = 6


def _combined_rows(year, month, day, wday, hour, minute):
    in_range = (year >= _YEAR0) & (year <= _YEAR0 + _YEARS - 1)
    yi = jnp.where(in_range, year - _YEAR0, year)
    yi = jnp.minimum(jnp.maximum(yi, 0), _YEARS - 1)
    mi = jnp.minimum(jnp.maximum(month - 1, 0), 11)
    di = jnp.minimum(jnp.maximum(day - 1, 0), 30)
    wi = jnp.minimum(jnp.maximum(wday, 0), 6)
    hi = jnp.minimum(jnp.maximum(hour, 0), 23)
    ni = jnp.minimum(jnp.maximum(lax.div(minute, 10), 0), 5)
    md = (mi * 31 + di) * _EMB
    wh = (wi * 24 + hi + _MD) * _EMB
    ym = (yi * 6 + ni + _MD + _WH) * _EMB
    return md, wh, ym


def _build_pairs(tbl_v, a_v, b_v, dst_off, nb, count):
    @plsc.parallel_loop(0, count, 1, unroll=2)
    def body(r):
        i = r // nb
        j = r - i * nb
        for k in range(_EMB // _L):
            va = a_v[pl.ds(i * _EMB + k * _L, _L)]
            vb = b_v[pl.ds(j * _EMB + k * _L, _L)]
            tbl_v[pl.ds((dst_off + r) * _EMB + k * _L, _L)] = va + vb


def _sc_lookup(xt_flat, npos, yw, mw, dw, wdw, hw, nw):
    per_w = npos // _NW
    nchunk = per_w // _CHUNK
    mesh = plsc.VectorSubcoreMesh(core_axis_name="c", subcore_axis_name="s")

    @functools.partial(
        pl.kernel,
        mesh=mesh,
        compiler_params=pltpu.CompilerParams(needs_layout_passes=False),
        out_type=jax.ShapeDtypeStruct((npos * _EMB,), jnp.float32),
        scratch_types=[
            pltpu.VMEM((_ROWS * _EMB,), jnp.float32),
            pltpu.VMEM((_YEARS * _EMB,), jnp.float32),
            pltpu.VMEM((12 * _EMB,), jnp.float32),
            pltpu.VMEM((31 * _EMB,), jnp.float32),
            pltpu.VMEM((7 * _EMB,), jnp.float32),
            pltpu.VMEM((24 * _EMB,), jnp.float32),
            pltpu.VMEM((6 * _EMB,), jnp.float32),
            pltpu.VMEM((_NF * _CHUNK,), jnp.int32),
            pltpu.VMEM((_NF * _CHUNK,), jnp.int32),
            pltpu.VMEM((_CHUNK * _EMB,), jnp.float32),
            pltpu.SemaphoreType.DMA,
            pltpu.SemaphoreType.DMA,
            pltpu.SemaphoreType.DMA,
        ],
    )
    def body(xt_hbm, yw_hbm, mw_hbm, dw_hbm, wdw_hbm, hw_hbm, nw_hbm,
             out_hbm, tbl_v, yv, mv, dv, wv, hv, nv, xa_v, xb_v, o_v,
             sem_xa, sem_xb, sem_o):
        wid = lax.axis_index("s") * _NC + lax.axis_index("c")
        base = wid * per_w

        def x_copies(t, x_v, sem):
            pos0 = base + t * _CHUNK
            return [pltpu.make_async_copy(
                        xt_hbm.at[pl.ds(f * npos + pos0, _CHUNK)],
                        x_v.at[pl.ds(f * _CHUNK, _CHUNK)], sem)
                    for f in range(_NF)]

        def o_copy(t):
            return pltpu.make_async_copy(
                o_v, out_hbm.at[pl.ds((base + t * _CHUNK) * _EMB,
                                      _CHUNK * _EMB)], sem_o)

        for c in x_copies(0, xa_v, sem_xa):
            c.start()

        tbl_loads = [pltpu.make_async_copy(src, dst, sem_o)
                     for src, dst in ((yw_hbm, yv), (mw_hbm, mv),
                                      (dw_hbm, dv), (wdw_hbm, wv),
                                      (hw_hbm, hv), (nw_hbm, nv))]
        for c in tbl_loads:
            c.start()
        for c in tbl_loads:
            c.wait()

        _build_pairs(tbl_v, mv, dv, 0, 31, _MD)
        _build_pairs(tbl_v, wv, hv, _MD, 24, _WH)
        _build_pairs(tbl_v, yv, nv, _MD + _WH, 6, _YM)

        lane = lax.iota(jnp.int32, _L)
        zero = jnp.int32(0)

        def fill_uniform(y0, m0, d0, w0, h0, n0):
            md, wh, ym = _combined_rows(y0, m0, d0, w0, h0, n0)
            rows = [tbl_v[pl.ds(md + k * _L, _L)]
                    + tbl_v[pl.ds(wh + k * _L, _L)]
                    + tbl_v[pl.ds(ym + k * _L, _L)]
                    for k in range(_EMB // _L)]

            @plsc.parallel_loop(0, _CHUNK, 1, unroll=4)
            def fill_body(p):
                for k in range(_EMB // _L):
                    o_v[pl.ds(p * _EMB + k * _L, _L)] = rows[k]

        def fill_general(x_v):
            for g in range(_CHUNK // _L):
                md, wh, ym = _combined_rows(
                    x_v[pl.ds(0 * _CHUNK + g * _L, _L)],
                    x_v[pl.ds(1 * _CHUNK + g * _L, _L)],
                    x_v[pl.ds(2 * _CHUNK + g * _L, _L)],
                    x_v[pl.ds(3 * _CHUNK + g * _L, _L)],
                    x_v[pl.ds(4 * _CHUNK + g * _L, _L)],
                    x_v[pl.ds(5 * _CHUNK + g * _L, _L)])
                ob = (g * _L + lane) * _EMB

                @plsc.parallel_loop(0, _EMB, 1, unroll=8)
                def col_body(c):
                    val = (plsc.load_gather(tbl_v, [md + c])
                           + plsc.load_gather(tbl_v, [wh + c])
                           + plsc.load_gather(tbl_v, [ym + c]))
                    plsc.store_scatter(o_v, [ob + c], val)

        def process(t, x_v, x_nxt, sem_nxt, carry):
            y0p, m0p, d0p, w0p, h0p, n0p, valid = carry

            @pl.when(t + 1 < nchunk)
            def _():
                for c in x_copies(t + 1, x_nxt, sem_nxt):
                    c.start()

            heads = [x_v[pl.ds(f * _CHUNK, _L)] for f in range(_NF)]
            y0 = heads[0][0]
            m0 = heads[1][0]
            d0 = heads[2][0]
            w0 = heads[3][0]
            h0 = heads[4][0]
            n0 = heads[5][0]
            scalars = (y0, m0, d0, w0, h0, n0)
            acc = lane < _L
            for f in range(_NF):
                acc = acc & (heads[f] == scalars[f])
            for g in range(1, _CHUNK // _L):
                for f in range(_NF):
                    acc = acc & (x_v[pl.ds(f * _CHUNK + g * _L, _L)]
                                 == scalars[f])
            uniform = jnp.all(acc)
            same = (uniform & (valid == 1)
                    & (y0 == y0p) & (m0 == m0p) & (d0 == d0p)
                    & (w0 == w0p) & (h0 == h0p) & (n0 == n0p))

            @pl.when(t > 0)
            def _():
                o_copy(t - 1).wait()

            def stale():
                lax.cond(uniform,
                         lambda: fill_uniform(y0, m0, d0, w0, h0, n0),
                         lambda: fill_general(x_v))

            lax.cond(same, lambda: None, stale)
            o_copy(t).start()
            return (y0, m0, d0, w0, h0, n0,
                    jnp.where(uniform, jnp.int32(1), zero))

        def pair_body(i, carry):
            t = i * 2
            for c in x_copies(t, xa_v, sem_xa):
                c.wait()
            carry = process(t, xa_v, xb_v, sem_xb, carry)
            for c in x_copies(t + 1, xb_v, sem_xb):
                c.wait()
            carry = process(t + 1, xb_v, xa_v, sem_xa, carry)
            return carry

        lax.fori_loop(0, nchunk // 2, pair_body,
                      (zero, zero, zero, zero, zero, zero, zero))
        o_copy(nchunk - 1).wait()

    return body(xt_flat, yw, mw, dw, wdw, hw, nw)


def kernel(x, year_w, month_w, day_w, weekday_w, hour_w, min_w):
    b, l, _ = x.shape
    xt_flat = jnp.transpose(x[:, :, :_NF], (2, 0, 1)).reshape(-1)
    out_flat = _sc_lookup(
        xt_flat, b * l,
        year_w.reshape(-1), month_w.reshape(-1), day_w.reshape(-1),
        weekday_w.reshape(-1), hour_w.reshape(-1), min_w.reshape(-1),
    )
    return out_flat.reshape(b, l, _EMB)


# speedup vs baseline: 1.1115x; 1.0097x over previous
import functools

import jax
import jax.numpy as jnp
from jax import lax
from jax.experimental import pallas as pl
from jax.experimental.pallas import tpu as pltpu
from jax.experimental.pallas import tpu_sc as plsc

_NC = 2
_NS = 16
_L = 16
_NW = _NC * _NS

_YEARS = 14
_YEAR0 = 2010
_EMB = 128

_MD = 12 * 31
_WH = 7 * 24
_YM = _YEARS * 6
_ROWS = _MD + _WH + _YM

_CHUNK = 256
_NF = 6


def _combined_rows(year, month, day, wday, hour, minute):
    in_range = (year >= _YEAR0) & (year <= _YEAR0 + _YEARS - 1)
    yi = jnp.where(in_range, year - _YEAR0, year)
    yi = jnp.minimum(jnp.maximum(yi, 0), _YEARS - 1)
    mi = jnp.minimum(jnp.maximum(month - 1, 0), 11)
    di = jnp.minimum(jnp.maximum(day - 1, 0), 30)
    wi = jnp.minimum(jnp.maximum(wday, 0), 6)
    hi = jnp.minimum(jnp.maximum(hour, 0), 23)
    ni = jnp.minimum(jnp.maximum(lax.div(minute, 10), 0), 5)
    md = (mi * 31 + di) * _EMB
    wh = (wi * 24 + hi + _MD) * _EMB
    ym = (yi * 6 + ni + _MD + _WH) * _EMB
    return md, wh, ym


def _build_pairs(tbl_v, a_v, b_v, dst_off, nb, count):
    @plsc.parallel_loop(0, count, 1, unroll=2)
    def body(r):
        i = r // nb
        j = r - i * nb
        for k in range(_EMB // _L):
            va = a_v[pl.ds(i * _EMB + k * _L, _L)]
            vb = b_v[pl.ds(j * _EMB + k * _L, _L)]
            tbl_v[pl.ds((dst_off + r) * _EMB + k * _L, _L)] = va + vb


def _sc_lookup(xt_flat, npos, yw, mw, dw, wdw, hw, nw):
    per_w = npos // _NW
    nchunk = per_w // _CHUNK
    mesh = plsc.VectorSubcoreMesh(core_axis_name="c", subcore_axis_name="s")

    @functools.partial(
        pl.kernel,
        mesh=mesh,
        compiler_params=pltpu.CompilerParams(needs_layout_passes=False),
        out_type=jax.ShapeDtypeStruct((npos * _EMB,), jnp.float32),
        scratch_types=[
            pltpu.VMEM((_ROWS * _EMB,), jnp.float32),
            pltpu.VMEM((_YEARS * _EMB,), jnp.float32),
            pltpu.VMEM((12 * _EMB,), jnp.float32),
            pltpu.VMEM((31 * _EMB,), jnp.float32),
            pltpu.VMEM((7 * _EMB,), jnp.float32),
            pltpu.VMEM((24 * _EMB,), jnp.float32),
            pltpu.VMEM((6 * _EMB,), jnp.float32),
            pltpu.VMEM((_NF * _CHUNK,), jnp.int32),
            pltpu.VMEM((_NF * _CHUNK,), jnp.int32),
            pltpu.VMEM((_CHUNK * _EMB,), jnp.float32),
            pltpu.SemaphoreType.DMA,
            pltpu.SemaphoreType.DMA,
            pltpu.SemaphoreType.DMA,
        ],
    )
    def body(xt_hbm, yw_hbm, mw_hbm, dw_hbm, wdw_hbm, hw_hbm, nw_hbm,
             out_hbm, tbl_v, yv, mv, dv, wv, hv, nv, xa_v, xb_v, o_v,
             sem_xa, sem_xb, sem_o):
        wid = lax.axis_index("s") * _NC + lax.axis_index("c")
        base = wid * per_w

        def x_copies(t, x_v, sem):
            pos0 = base + t * _CHUNK
            return [pltpu.make_async_copy(
                        xt_hbm.at[pl.ds(f * npos + pos0, _CHUNK)],
                        x_v.at[pl.ds(f * _CHUNK, _CHUNK)], sem)
                    for f in range(_NF)]

        def o_copy(t):
            return pltpu.make_async_copy(
                o_v, out_hbm.at[pl.ds((base + t * _CHUNK) * _EMB,
                                      _CHUNK * _EMB)], sem_o)

        for c in x_copies(0, xa_v, sem_xa):
            c.start()

        tbl_loads = [pltpu.make_async_copy(src, dst, sem_o)
                     for src, dst in ((yw_hbm, yv), (mw_hbm, mv),
                                      (dw_hbm, dv), (wdw_hbm, wv),
                                      (hw_hbm, hv), (nw_hbm, nv))]
        for c in tbl_loads:
            c.start()
        for c in tbl_loads:
            c.wait()

        _build_pairs(tbl_v, mv, dv, 0, 31, _MD)
        _build_pairs(tbl_v, wv, hv, _MD, 24, _WH)
        _build_pairs(tbl_v, yv, nv, _MD + _WH, 6, _YM)

        lane = lax.iota(jnp.int32, _L)
        zero = jnp.int32(0)

        def fill_uniform(y0, m0, d0, w0, h0, n0):
            md, wh, ym = _combined_rows(y0, m0, d0, w0, h0, n0)
            rows = [tbl_v[pl.ds(md + k * _L, _L)]
                    + tbl_v[pl.ds(wh + k * _L, _L)]
                    + tbl_v[pl.ds(ym + k * _L, _L)]
                    for k in range(_EMB // _L)]

            @plsc.parallel_loop(0, _CHUNK, 1, unroll=4)
            def fill_body(p):
                for k in range(_EMB // _L):
                    o_v[pl.ds(p * _EMB + k * _L, _L)] = rows[k]

        def fill_general(x_v):
            for g in range(_CHUNK // _L):
                md, wh, ym = _combined_rows(
                    x_v[pl.ds(0 * _CHUNK + g * _L, _L)],
                    x_v[pl.ds(1 * _CHUNK + g * _L, _L)],
                    x_v[pl.ds(2 * _CHUNK + g * _L, _L)],
                    x_v[pl.ds(3 * _CHUNK + g * _L, _L)],
                    x_v[pl.ds(4 * _CHUNK + g * _L, _L)],
                    x_v[pl.ds(5 * _CHUNK + g * _L, _L)])
                ob = (g * _L + lane) * _EMB

                @plsc.parallel_loop(0, _EMB, 1, unroll=8)
                def col_body(c):
                    val = (plsc.load_gather(tbl_v, [md + c])
                           + plsc.load_gather(tbl_v, [wh + c])
                           + plsc.load_gather(tbl_v, [ym + c]))
                    plsc.store_scatter(o_v, [ob + c], val)

        def process(t, x_v, x_nxt, sem_nxt, carry):
            y0p, m0p, d0p, w0p, h0p, n0p, valid = carry

            @pl.when(t + 1 < nchunk)
            def _():
                for c in x_copies(t + 1, x_nxt, sem_nxt):
                    c.start()

            heads = [x_v[pl.ds(f * _CHUNK, _L)] for f in range(_NF)]
            y0 = heads[0][0]
            m0 = heads[1][0]
            d0 = heads[2][0]
            w0 = heads[3][0]
            h0 = heads[4][0]
            n0 = heads[5][0]
            scalars = (y0, m0, d0, w0, h0, n0)
            acc = lane < _L
            for f in range(_NF):
                acc = acc & (heads[f] == scalars[f])
            for g in range(1, _CHUNK // _L):
                for f in range(_NF):
                    acc = acc & (x_v[pl.ds(f * _CHUNK + g * _L, _L)]
                                 == scalars[f])
            uniform = jnp.all(acc)
            same = (uniform & (valid == 1)
                    & (y0 == y0p) & (m0 == m0p) & (d0 == d0p)
                    & (w0 == w0p) & (h0 == h0p) & (n0 == n0p))

            @pl.when(t > 0)
            def _():
                o_copy(t - 1).wait()

            def stale():
                lax.cond(uniform,
                         lambda: fill_uniform(y0, m0, d0, w0, h0, n0),
                         lambda: fill_general(x_v))

            lax.cond(same, lambda: None, stale)
            o_copy(t).start()
            return (y0, m0, d0, w0, h0, n0,
                    jnp.where(uniform, jnp.int32(1), zero))

        def pair_body(i, carry):
            t = i * 2
            for c in x_copies(t, xa_v, sem_xa):
                c.wait()
            carry = process(t, xa_v, xb_v, sem_xb, carry)
            for c in x_copies(t + 1, xb_v, sem_xb):
                c.wait()
            carry = process(t + 1, xb_v, xa_v, sem_xa, carry)
            return carry

        lax.fori_loop(0, nchunk // 2, pair_body,
                      (zero, zero, zero, zero, zero, zero, zero))
        o_copy(nchunk - 1).wait()

    return body(xt_flat, yw, mw, dw, wdw, hw, nw)


def kernel(x, year_w, month_w, day_w, weekday_w, hour_w, min_w):
    b, l, _ = x.shape
    xt_flat = jnp.transpose(x, (2, 0, 1)).reshape(-1)
    out_flat = _sc_lookup(
        xt_flat, b * l,
        year_w.reshape(-1), month_w.reshape(-1), day_w.reshape(-1),
        weekday_w.reshape(-1), hour_w.reshape(-1), min_w.reshape(-1),
    )
    return out_flat.reshape(b, l, _EMB)
